# baseline (speedup 1.0000x reference)
import functools

import jax
import jax.numpy as jnp
from jax import lax
from jax.experimental import pallas as pl
from jax.experimental.pallas import tpu as pltpu
from jax.experimental.pallas import tpu_sc as plsc

VOCAB = 1000000
EMBED = 64
BATCH = 4096
HALF = BATCH // 2

_NC = 2
_NS = 16
_NW = _NC * _NS
_L = 16
_CH = 4


def _gather_one_table(wt_hbm, idx_hbm, out_hbm, nrows, base, idx_v, off_v,
                      lane_v, bufs, rows_v, sem):
    pltpu.sync_copy(idx_hbm.at[pl.ds(base, nrows)],
                    idx_v.at[pl.ds(0, nrows)])
    lanes16 = lax.iota(jnp.int32, _L)

    for i in range(nrows // _L):
        v = idx_v[pl.ds(i * _L, _L)]
        off_v[pl.ds(i * _L, _L)] = (v >> 7) << 7
        lane_v[pl.ds(i * _L, _L)] = v & 127

    def fire(j, c):
        vo = off_v[pl.ds(j * 32 + (c // 4) * _L, _L)]
        for k in range(_CH):
            off = pl.multiple_of(vo[(c % 4) * _CH + k], 128)
            pltpu.async_copy(wt_hbm.at[:, pl.ds(off, 128)],
                             bufs.at[(c % 2) * _CH + k], sem)

    def run(j, _):
        fire(j, 0)
        for c in range(8):
            if c + 1 < 8:
                fire(j, c + 1)
            for k in range(_CH):
                pltpu.make_async_copy(wt_hbm.at[:, pl.ds(0, 128)],
                                      bufs.at[(c % 2) * _CH + k], sem).wait()
            vl = lane_v[pl.ds(j * 32 + (c // 4) * _L, _L)]
            for k in range(_CH):
                i = j * 32 + c * _CH + k
                lane = jnp.full((_L,), vl[(c % 4) * _CH + k], jnp.int32)
                slot = jnp.full((_L,), (c % 2) * _CH + k, jnp.int32)
                for q in range(EMBED // _L):
                    vals = plsc.load_gather(
                        bufs, [slot, lanes16 + q * _L, lane])
                    rows_v[i, pl.ds(q * _L, _L)] = vals
        return ()

    lax.fori_loop(0, nrows // 32, run, ())
    pltpu.sync_copy(rows_v.at[pl.ds(0, nrows)],
                    out_hbm.at[pl.ds(base, nrows)])


_SCRATCH = [
    pltpu.VMEM((BATCH // _NW,), jnp.int32),
    pltpu.VMEM((BATCH // _NW,), jnp.int32),
    pltpu.VMEM((BATCH // _NW,), jnp.int32),
    pltpu.VMEM((2 * _CH, EMBED, 128), jnp.float32),
    pltpu.VMEM((BATCH // _NW, EMBED), jnp.float32),
    pltpu.SemaphoreType.DMA,
]
_MESH = plsc.VectorSubcoreMesh(core_axis_name="c", subcore_axis_name="s")


@functools.partial(
    pl.kernel,
    out_type=(
        jax.ShapeDtypeStruct((BATCH, EMBED), jnp.float32),
        jax.ShapeDtypeStruct((HALF, EMBED), jnp.float32),
    ),
    mesh=_MESH,
    compiler_params=pltpu.CompilerParams(needs_layout_passes=False),
    scratch_types=_SCRATCH,
)
def _sc_gather_main(wct_hbm, ci_hbm, wxt_hbm, xi1_hbm, out_c, out_x1,
                    idx_v, off_v, lane_v, bufs, rows_v, sem):
    wid = lax.axis_index("s") * _NC + lax.axis_index("c")
    _gather_one_table(wct_hbm, ci_hbm, out_c, BATCH // _NW,
                      wid * (BATCH // _NW), idx_v, off_v, lane_v, bufs,
                      rows_v, sem)
    _gather_one_table(wxt_hbm, xi1_hbm, out_x1, HALF // _NW,
                      wid * (HALF // _NW), idx_v, off_v, lane_v, bufs,
                      rows_v, sem)


@functools.partial(
    pl.kernel,
    out_type=jax.ShapeDtypeStruct((HALF, EMBED), jnp.float32),
    mesh=_MESH,
    compiler_params=pltpu.CompilerParams(needs_layout_passes=False),
    scratch_types=_SCRATCH,
)
def _sc_gather_rest(wxt_hbm, xi2_hbm, out_x2,
                    idx_v, off_v, lane_v, bufs, rows_v, sem):
    wid = lax.axis_index("s") * _NC + lax.axis_index("c")
    _gather_one_table(wxt_hbm, xi2_hbm, out_x2, HALF // _NW,
                      wid * (HALF // _NW), idx_v, off_v, lane_v, bufs,
                      rows_v, sem)


_TM = 512


def _mm_body(a_ref, b_ref, o_ref):
    a = a_ref[...].astype(jnp.bfloat16)
    b = b_ref[...].astype(jnp.bfloat16)
    o_ref[...] = lax.dot_general(
        a, b,
        dimension_numbers=(((1,), (1,)), ((), ())),
        preferred_element_type=jnp.float32,
    )


def _mm_body_alias(a_ref, b_ref, s_ref, o_ref):
    del s_ref
    _mm_body(a_ref, b_ref, o_ref)


def _mm_half(a, b_half, col, scores=None):
    args = (a, b_half) if scores is None else (a, b_half, scores)
    kwargs = {} if scores is None else {"input_output_aliases": {2: 0}}
    extra_spec = [] if scores is None else [pl.BlockSpec(memory_space=pltpu.MemorySpace.HBM)]
    return pl.pallas_call(
        _mm_body if scores is None else _mm_body_alias,
        grid=(BATCH // _TM,),
        in_specs=[
            pl.BlockSpec((_TM, EMBED), lambda i: (i, 0)),
            pl.BlockSpec((HALF, EMBED), lambda i: (0, 0)),
        ] + extra_spec,
        out_specs=pl.BlockSpec((_TM, HALF), lambda i: (i, col)),
        out_shape=jax.ShapeDtypeStruct((BATCH, BATCH), jnp.float32),
        **kwargs,
    )(*args)


def kernel(center_word, context_word, W_center, W_context):
    ci = center_word.astype(jnp.int32)
    xi = context_word.astype(jnp.int32)
    wct, wxt = W_center.T, W_context.T
    ce, cx1 = _sc_gather_main(wct, ci, wxt, xi[:HALF])
    cx2 = _sc_gather_rest(wxt, xi[HALF:])
    s = _mm_half(ce, cx1, 0)
    return _mm_half(ce, cx2, 1, s)


# speedup vs baseline: 1.0281x; 1.0281x over previous
import functools

import jax
import jax.numpy as jnp
from jax import lax
from jax.experimental import pallas as pl
from jax.experimental.pallas import tpu as pltpu
from jax.experimental.pallas import tpu_sc as plsc

VOCAB = 1000000
EMBED = 64
BATCH = 4096

_NC = 2
_NS = 16
_NW = _NC * _NS
_BPW = BATCH // _NW
_L = 16
_CH = 4
_NCH = 8


def _gather_one_table(wt_hbm, idx_hbm, out_hbm, base, idx_v, off_v, lane_v,
                      bufs, rows_v, sem):
    pltpu.sync_copy(idx_hbm.at[pl.ds(base, _BPW)], idx_v)
    lanes16 = lax.iota(jnp.int32, _L)

    for i in range(_BPW // _L):
        v = idx_v[pl.ds(i * _L, _L)]
        off_v[pl.ds(i * _L, _L)] = (v >> 7) << 7
        lane_v[pl.ds(i * _L, _L)] = v & 127

    def fire(j, c):
        vo = off_v[pl.ds(j * 32 + (c // 4) * _L, _L)]
        for k in range(_CH):
            off = pl.multiple_of(vo[(c % 4) * _CH + k], 128)
            pltpu.async_copy(wt_hbm.at[:, pl.ds(off, 128)],
                             bufs.at[(c % 2) * _CH + k], sem)

    def run(j, _):
        fire(j, 0)
        for c in range(_NCH):
            if c + 1 < _NCH:
                fire(j, c + 1)
            for k in range(_CH):
                pltpu.make_async_copy(wt_hbm.at[:, pl.ds(0, 128)],
                                      bufs.at[(c % 2) * _CH + k], sem).wait()
            vl = lane_v[pl.ds(j * 32 + (c // 4) * _L, _L)]
            for k in range(_CH):
                i = j * 32 + c * _CH + k
                lane = jnp.full((_L,), vl[(c % 4) * _CH + k], jnp.int32)
                slot = jnp.full((_L,), (c % 2) * _CH + k, jnp.int32)
                for q in range(EMBED // _L):
                    vals = plsc.load_gather(
                        bufs, [slot, lanes16 + q * _L, lane])
                    rows_v[i, pl.ds(q * _L, _L)] = vals
        return ()

    lax.fori_loop(0, _BPW // 32, run, ())
    pltpu.sync_copy(rows_v, out_hbm.at[pl.ds(base, _BPW)])


@functools.partial(
    pl.kernel,
    out_type=(
        jax.ShapeDtypeStruct((BATCH, EMBED), jnp.float32),
        jax.ShapeDtypeStruct((BATCH, EMBED), jnp.float32),
    ),
    mesh=plsc.VectorSubcoreMesh(core_axis_name="c", subcore_axis_name="s"),
    compiler_params=pltpu.CompilerParams(needs_layout_passes=False),
    scratch_types=[
        pltpu.VMEM((_BPW,), jnp.int32),
        pltpu.VMEM((_BPW,), jnp.int32),
        pltpu.VMEM((_BPW,), jnp.int32),
        pltpu.VMEM((2 * _CH, EMBED, 128), jnp.float32),
        pltpu.VMEM((_BPW, EMBED), jnp.float32),
        pltpu.SemaphoreType.DMA,
    ],
)
def _sc_gather(wct_hbm, ci_hbm, wxt_hbm, xi_hbm, out_c, out_x,
               idx_v, off_v, lane_v, bufs, rows_v, sem):
    wid = lax.axis_index("s") * _NC + lax.axis_index("c")
    base = wid * _BPW
    _gather_one_table(wct_hbm, ci_hbm, out_c, base, idx_v, off_v, lane_v,
                      bufs, rows_v, sem)
    _gather_one_table(wxt_hbm, xi_hbm, out_x, base, idx_v, off_v, lane_v,
                      bufs, rows_v, sem)


_TM = 1024


def _mm_body(a_ref, b_ref, o_ref):
    a = a_ref[...].astype(jnp.bfloat16)
    b = b_ref[...].astype(jnp.bfloat16)
    o_ref[...] = lax.dot_general(
        a, b,
        dimension_numbers=(((1,), (1,)), ((), ())),
        preferred_element_type=jnp.float32,
    )


def _tc_matmul(a, b):
    return pl.pallas_call(
        _mm_body,
        grid=(BATCH // _TM,),
        in_specs=[
            pl.BlockSpec((_TM, EMBED), lambda i: (i, 0)),
            pl.BlockSpec((BATCH, EMBED), lambda i: (0, 0)),
        ],
        out_specs=pl.BlockSpec((_TM, BATCH), lambda i: (i, 0)),
        out_shape=jax.ShapeDtypeStruct((BATCH, BATCH), jnp.float32),
    )(a, b)


def kernel(center_word, context_word, W_center, W_context):
    ce, cx = _sc_gather(W_center.T, center_word.astype(jnp.int32),
                        W_context.T, context_word.astype(jnp.int32))
    return _tc_matmul(ce, cx)


# speedup vs baseline: 1.0372x; 1.0089x over previous
import functools

import jax
import jax.numpy as jnp
from jax import lax
from jax.experimental import pallas as pl
from jax.experimental.pallas import tpu as pltpu
from jax.experimental.pallas import tpu_sc as plsc

VOCAB = 1000000
EMBED = 64
BATCH = 4096

_NC = 2
_NS = 16
_NW = _NC * _NS
_BPW = BATCH // _NW
_L = 16
_CH = 4
_NCH = 8


def _gather_one_table(wt_hbm, idx_hbm, out_hbm, base, idx_v, off_v, lane_v,
                      bufs, rows_v, sem):
    pltpu.sync_copy(idx_hbm.at[pl.ds(base, _BPW)], idx_v)
    lanes16 = lax.iota(jnp.int32, _L)

    for i in range(_BPW // _L):
        v = idx_v[pl.ds(i * _L, _L)]
        off_v[pl.ds(i * _L, _L)] = (v >> 7) << 7
        lane_v[pl.ds(i * _L, _L)] = v & 127

    def fire(j, c):
        vo = off_v[pl.ds(j * 32 + (c // 4) * _L, _L)]
        for k in range(_CH):
            off = pl.multiple_of(vo[(c % 4) * _CH + k], 128)
            pltpu.async_copy(wt_hbm.at[:, pl.ds(off, 128)],
                             bufs.at[(c % 2) * _CH + k], sem)

    def run(j, _):
        fire(j, 0)
        for c in range(_NCH):
            if c + 1 < _NCH:
                fire(j, c + 1)
            for k in range(_CH):
                pltpu.make_async_copy(wt_hbm.at[:, pl.ds(0, 128)],
                                      bufs.at[(c % 2) * _CH + k], sem).wait()
            vl = lane_v[pl.ds(j * 32 + (c // 4) * _L, _L)]
            for k in range(_CH):
                i = j * 32 + c * _CH + k
                lane = jnp.full((_L,), vl[(c % 4) * _CH + k], jnp.int32)
                slot = jnp.full((_L,), (c % 2) * _CH + k, jnp.int32)
                for q in range(EMBED // _L):
                    vals = plsc.load_gather(
                        bufs, [slot, lanes16 + q * _L, lane])
                    rows_v[i, pl.ds(q * _L, _L)] = vals
        return ()

    lax.fori_loop(0, _BPW // 32, run, ())
    pltpu.sync_copy(rows_v, out_hbm.at[pl.ds(base, _BPW)])


@functools.partial(
    pl.kernel,
    out_type=(
        jax.ShapeDtypeStruct((BATCH, EMBED), jnp.float32),
        jax.ShapeDtypeStruct((BATCH, EMBED), jnp.float32),
    ),
    mesh=plsc.VectorSubcoreMesh(core_axis_name="c", subcore_axis_name="s"),
    compiler_params=pltpu.CompilerParams(needs_layout_passes=False),
    scratch_types=[
        pltpu.VMEM((_BPW,), jnp.int32),
        pltpu.VMEM((_BPW,), jnp.int32),
        pltpu.VMEM((_BPW,), jnp.int32),
        pltpu.VMEM((2 * _CH, EMBED, 128), jnp.float32),
        pltpu.VMEM((_BPW, EMBED), jnp.float32),
        pltpu.SemaphoreType.DMA,
    ],
)
def _sc_gather(wct_hbm, ci_hbm, wxt_hbm, xi_hbm, out_c, out_x,
               idx_v, off_v, lane_v, bufs, rows_v, sem):
    wid = lax.axis_index("s") * _NC + lax.axis_index("c")
    base = wid * _BPW
    _gather_one_table(wct_hbm, ci_hbm, out_c, base, idx_v, off_v, lane_v,
                      bufs, rows_v, sem)
    _gather_one_table(wxt_hbm, xi_hbm, out_x, base, idx_v, off_v, lane_v,
                      bufs, rows_v, sem)


_TM = 512


def _mm_body(a_ref, b_ref, o_ref):
    a = a_ref[...].astype(jnp.bfloat16)
    b = b_ref[...].astype(jnp.bfloat16)
    o_ref[...] = lax.dot_general(
        a, b,
        dimension_numbers=(((1,), (1,)), ((), ())),
        preferred_element_type=jnp.float32,
    )


def _tc_matmul(a, b):
    return pl.pallas_call(
        _mm_body,
        grid=(BATCH // _TM,),
        in_specs=[
            pl.BlockSpec((_TM, EMBED), lambda i: (i, 0)),
            pl.BlockSpec((BATCH, EMBED), lambda i: (0, 0)),
        ],
        out_specs=pl.BlockSpec((_TM, BATCH), lambda i: (i, 0)),
        out_shape=jax.ShapeDtypeStruct((BATCH, BATCH), jnp.float32),
    )(a, b)


def kernel(center_word, context_word, W_center, W_context):
    ce, cx = _sc_gather(W_center.T, center_word.astype(jnp.int32),
                        W_context.T, context_word.astype(jnp.int32))
    return _tc_matmul(ce, cx)
